# baseline (speedup 1.0000x reference)
import functools

import jax
import jax.numpy as jnp
from jax import lax
from jax.experimental import pallas as pl
from jax.experimental.pallas import tpu as pltpu
from jax.experimental.pallas import tpu_sc as plsc

N_NODES = 10000
EMB = 128
HID = 128
FF = 400
N_USED = 9000

NC = 2
NS = 16
NW = NC * NS
CH = 128
NCH0 = 99
NCH1 = 60
TOTCH = NS * (NCH0 + NCH1)
E_PAD = TOTCH * CH
N_ACC = 9216
ROWS_PER_TILE = N_ACC // NS
CPR = 96


def _sc_agg(hyper_hbm, meta_hbm, w_hbm, emb_hbm, out_hbm,
            ma, mb, mc, wa, wb, wc, ent_a, ent_b, ent_c, rows_v,
            sema, semb, semc, agg_sh):
    c = lax.axis_index("c")
    s = lax.axis_index("s")
    nch = jnp.where(c == 0, NCH0, NCH1)
    cb = jnp.where(c == 0, s * NCH0, NS * NCH0 + s * NCH1)

    def _zb(i, _):
        r = i // (EMB // 16)
        f = i % (EMB // 16)
        rows_v[r, pl.ds(f * 16, 16)] = jnp.zeros((16,), jnp.float32)
        return 0
    lax.fori_loop(0, CPR * (EMB // 16), _zb, 0)
    row0 = s * ROWS_PER_TILE
    zslot = rows_v.at[pl.ds(0, CPR)]
    for k in range(ROWS_PER_TILE // CPR):
        pltpu.sync_copy(zslot, agg_sh.at[pl.ds(row0 + k * CPR, CPR)])
    plsc.subcore_barrier()

    def _scale(base, wv):
        def _sgrp(j, _):
            w16 = wv[pl.ds(j * 16, 16)]
            for k in range(16):
                wspl = lax.gather(
                    w16, jnp.full((16, 1), k, jnp.int32),
                    lax.GatherDimensionNumbers(
                        offset_dims=(), collapsed_slice_dims=(0,),
                        start_index_map=(0,)),
                    slice_sizes=(1,),
                    mode=lax.GatherScatterMode.PROMISE_IN_BOUNDS)
                e = base + j * 16 + k
                for f in range(EMB // 16):
                    rows_v[e, pl.ds(f * 16, 16)] = (
                        rows_v[e, pl.ds(f * 16, 16)] * wspl)
            return 0
        lax.fori_loop(0, CH // 16, _sgrp, 0)

    slot = lambda p: rows_v.at[pl.ds(p * CH, CH)]

    def _stage(x, m, wv, e, sem, p):
        pltpu.sync_copy(meta_hbm.at[cb + x], m)
        pltpu.sync_copy(w_hbm.at[cb + x], wv)
        pltpu.sync_copy(hyper_hbm.at[m.at[0]], e)
        pltpu.async_copy(emb_hbm.at[e], slot(p), sem)

    def _proc(m, wv, sem, p):
        pltpu.make_async_copy(emb_hbm.at[pl.ds(0, CH)], slot(p),
                              sem).wait()
        _scale(p * CH, wv)
        pltpu.sync_copy(slot(p), agg_sh.at[m.at[1]], add=True)

    _stage(0, ma, wa, ent_a, sema, 0)
    _stage(1, mb, wb, ent_b, semb, 1)

    def _trip(j, _):
        a = 3 * j
        _stage(a + 2, mc, wc, ent_c, semc, 2)
        _proc(ma, wa, sema, 0)

        @pl.when(a + 3 < nch)
        def _():
            _stage(a + 3, ma, wa, ent_a, sema, 0)

        _proc(mb, wb, semb, 1)

        @pl.when(a + 4 < nch)
        def _():
            _stage(a + 4, mb, wb, ent_b, semb, 1)

        _proc(mc, wc, semc, 2)
        return 0
    lax.fori_loop(0, nch // 3, _trip, 0)

    plsc.subcore_barrier()
    wslot = rows_v.at[pl.ds(0, CPR)]
    for k in range(ROWS_PER_TILE // CPR):
        r = row0 + k * CPR
        pltpu.sync_copy(agg_sh.at[pl.ds(r, CPR)], wslot)
        pltpu.sync_copy(wslot, out_hbm.at[c, pl.ds(r, CPR)])


def _sc_aggregate(hyperneigh, meta, w, emb):
    mesh = plsc.VectorSubcoreMesh(core_axis_name="c", subcore_axis_name="s")
    return pl.kernel(
        _sc_agg,
        mesh=mesh,
        out_type=jax.ShapeDtypeStruct((NC, N_ACC, EMB), jnp.float32),
        scratch_types=[
            pltpu.VMEM((2, CH), jnp.int32),
            pltpu.VMEM((2, CH), jnp.int32),
            pltpu.VMEM((2, CH), jnp.int32),
            pltpu.VMEM((CH,), jnp.float32),
            pltpu.VMEM((CH,), jnp.float32),
            pltpu.VMEM((CH,), jnp.float32),
            pltpu.VMEM((CH,), jnp.int32),
            pltpu.VMEM((CH,), jnp.int32),
            pltpu.VMEM((CH,), jnp.int32),
            pltpu.VMEM((3 * CH, EMB), jnp.float32),
            pltpu.SemaphoreType.DMA,
            pltpu.SemaphoreType.DMA,
            pltpu.SemaphoreType.DMA,
            pltpu.VMEM_SHARED((N_ACC, EMB), jnp.float32),
        ],
    )(hyperneigh, meta, w, emb)


def _ln(x, g, b):
    m = jnp.mean(x, axis=-1, keepdims=True)
    v = jnp.mean((x - m) ** 2, axis=-1, keepdims=True)
    return (x - m) * lax.rsqrt(v + 1e-5) * g + b


BLK = 1000


def _tc_body(p0, p1, W1, b1, Wv, bv, Wo, bo, l1g, l1b, Wf1, bf1, Wf2, bf2,
             l2g, l2b, fW, fb, out_ref, acc_ref):
    b = pl.program_id(0)
    agg = p0[0] + p1[0]
    o = jnp.dot(agg, W1[...], preferred_element_type=jnp.float32) + b1[...]
    ct = (((1,), (1,)), ((), ()))
    a = lax.dot_general(o, Wv[...], ct, preferred_element_type=jnp.float32) + bv[...]
    a = lax.dot_general(a, Wo[...], ct, preferred_element_type=jnp.float32) + bo[...]
    y = _ln(o + a, l1g[...], l1b[...])
    f = lax.dot_general(y, Wf1[...], ct, preferred_element_type=jnp.float32) + bf1[...]
    f = jnp.maximum(f, 0.0)
    f = lax.dot_general(f, Wf2[...], ct, preferred_element_type=jnp.float32) + bf2[...]
    z = _ln(y + f, l2g[...], l2b[...])
    colsum = jnp.sum(z, axis=0, keepdims=True)
    coef = jnp.where(b < 4, 1.0 / 4000.0,
                     jnp.where(b < 7, -1.0 / 3000.0, 1.0 / 2000.0))

    @pl.when(b == 0)
    def _():
        acc_ref[...] = jnp.zeros_like(acc_ref)

    acc_ref[...] += coef.astype(jnp.float32) * colsum

    @pl.when(b == (N_USED // BLK) - 1)
    def _():
        h = acc_ref[...]
        r = lax.dot_general(h, fW[...], ct, preferred_element_type=jnp.float32) + fb[...]
        out_ref[...] = jnp.maximum(r, 0.0)[None]


def _tc_dense(partials, W1, b1, Wv, bv, Wo, bo, l1g, l1b, Wf1, bf1, Wf2, bf2,
              l2g, l2b, fW, fb):
    n_blk = N_USED // BLK
    full = lambda arr: pl.BlockSpec(arr.shape, lambda b: (0,) * arr.ndim)
    w_specs = [full(a) for a in (W1, b1, Wv, bv, Wo, bo, l1g, l1b,
                                 Wf1, bf1, Wf2, bf2, l2g, l2b, fW, fb)]
    return pl.pallas_call(
        _tc_body,
        grid=(n_blk,),
        in_specs=[
            pl.BlockSpec((1, BLK, EMB), lambda b: (0, b, 0)),
            pl.BlockSpec((1, BLK, EMB), lambda b: (1, b, 0)),
            *w_specs,
        ],
        out_specs=pl.BlockSpec((1, 1, HID), lambda b: (0, 0, 0)),
        out_shape=jax.ShapeDtypeStruct((1, 1, HID), jnp.float32),
        scratch_shapes=[pltpu.VMEM((1, HID), jnp.float32)],
    )(partials, partials, W1, b1, Wv, bv, Wo, bo, l1g, l1b,
      Wf1, bf1, Wf2, bf2, l2g, l2b, fW, fb)


def _edge_layout(src, dst, w):
    meta = jnp.stack([src.reshape(TOTCH, CH), dst.reshape(TOTCH, CH)],
                     axis=1)
    return meta, w.reshape(TOTCH, CH)


def kernel(hyperneigh, edge_index, edge_weight, acc_len, rej_len, friend_len,
           emb, W1, b1, Wq, bq, Wk, bk, Wv, bv, Wo, bo, ln1_g, ln1_b,
           Wf1, bf1, Wf2, bf2, ln2_g, ln2_b, fc1_W, fc1_b):
    src = edge_index[0]
    dst = edge_index[1]
    pad = E_PAD - src.shape[0]
    src = jnp.concatenate([src, jnp.zeros((pad,), src.dtype)])
    dst = jnp.minimum(dst, N_USED)
    dst = jnp.concatenate([dst, jnp.zeros((pad,), dst.dtype)])
    w = jnp.concatenate([edge_weight, jnp.zeros((pad,), edge_weight.dtype)])
    meta, wm = _edge_layout(src, dst, w)

    partials = _sc_aggregate(hyperneigh, meta, wm, emb)

    r2 = lambda v: v.reshape(1, -1)
    return _tc_dense(partials, W1, r2(b1), Wv, r2(bv), Wo, r2(bo),
                     r2(ln1_g), r2(ln1_b), Wf1, r2(bf1), Wf2, r2(bf2),
                     r2(ln2_g), r2(ln2_b), fc1_W, r2(fc1_b))


# speedup vs baseline: 1.0266x; 1.0266x over previous
import functools

import jax
import jax.numpy as jnp
from jax import lax
from jax.experimental import pallas as pl
from jax.experimental.pallas import tpu as pltpu
from jax.experimental.pallas import tpu_sc as plsc

N_NODES = 10000
EMB = 128
HID = 128
FF = 400
N_USED = 9000

NC = 2
NS = 16
NW = NC * NS
CH = 128
NCH0 = 102
NCH1 = 57
TOTCH = NS * (NCH0 + NCH1)
E_PAD = TOTCH * CH
N_ACC = 9216
ROWS_PER_TILE = N_ACC // NS
CPR = 96


def _sc_agg(hyper_hbm, meta_hbm, w_hbm, emb_hbm, out_hbm,
            ma, mb, mc, wa, wb, wc, ent_a, ent_b, ent_c, rows_v,
            sema, semb, semc, agg_sh):
    c = lax.axis_index("c")
    s = lax.axis_index("s")
    nch = jnp.where(c == 0, NCH0, NCH1)
    cb = jnp.where(c == 0, s * NCH0, NS * NCH0 + s * NCH1)

    def _zb(i, _):
        r = i // (EMB // 16)
        f = i % (EMB // 16)
        rows_v[r, pl.ds(f * 16, 16)] = jnp.zeros((16,), jnp.float32)
        return 0
    lax.fori_loop(0, CPR * (EMB // 16), _zb, 0)
    row0 = s * ROWS_PER_TILE
    zslot = rows_v.at[pl.ds(0, CPR)]
    for k in range(ROWS_PER_TILE // CPR):
        pltpu.sync_copy(zslot, agg_sh.at[pl.ds(row0 + k * CPR, CPR)])
    plsc.subcore_barrier()

    def _scale(base, wv):
        def _sgrp(j, _):
            w16 = wv[pl.ds(j * 16, 16)]
            for k in range(16):
                wspl = lax.gather(
                    w16, jnp.full((16, 1), k, jnp.int32),
                    lax.GatherDimensionNumbers(
                        offset_dims=(), collapsed_slice_dims=(0,),
                        start_index_map=(0,)),
                    slice_sizes=(1,),
                    mode=lax.GatherScatterMode.PROMISE_IN_BOUNDS)
                e = base + j * 16 + k
                for f in range(EMB // 16):
                    rows_v[e, pl.ds(f * 16, 16)] = (
                        rows_v[e, pl.ds(f * 16, 16)] * wspl)
            return 0
        lax.fori_loop(0, CH // 16, _sgrp, 0)

    slot = lambda p: rows_v.at[pl.ds(p * CH, CH)]

    def _stage(x, m, wv, e, sem, p):
        pltpu.sync_copy(meta_hbm.at[cb + x], m)
        pltpu.sync_copy(w_hbm.at[cb + x], wv)
        pltpu.sync_copy(hyper_hbm.at[m.at[0]], e)
        pltpu.async_copy(emb_hbm.at[e], slot(p), sem)

    def _proc(m, wv, sem, p):
        pltpu.make_async_copy(emb_hbm.at[pl.ds(0, CH)], slot(p),
                              sem).wait()
        _scale(p * CH, wv)
        pltpu.sync_copy(slot(p), agg_sh.at[m.at[1]], add=True)

    _stage(0, ma, wa, ent_a, sema, 0)
    _stage(1, mb, wb, ent_b, semb, 1)

    def _trip(j, _):
        a = 3 * j
        _stage(a + 2, mc, wc, ent_c, semc, 2)
        _proc(ma, wa, sema, 0)

        @pl.when(a + 3 < nch)
        def _():
            _stage(a + 3, ma, wa, ent_a, sema, 0)

        _proc(mb, wb, semb, 1)

        @pl.when(a + 4 < nch)
        def _():
            _stage(a + 4, mb, wb, ent_b, semb, 1)

        _proc(mc, wc, semc, 2)
        return 0
    lax.fori_loop(0, nch // 3, _trip, 0)

    plsc.subcore_barrier()
    wslot = rows_v.at[pl.ds(0, CPR)]
    for k in range(ROWS_PER_TILE // CPR):
        r = row0 + k * CPR
        pltpu.sync_copy(agg_sh.at[pl.ds(r, CPR)], wslot)
        pltpu.sync_copy(wslot, out_hbm.at[c, pl.ds(r, CPR)])


def _sc_aggregate(hyperneigh, meta, w, emb):
    mesh = plsc.VectorSubcoreMesh(core_axis_name="c", subcore_axis_name="s")
    return pl.kernel(
        _sc_agg,
        mesh=mesh,
        out_type=jax.ShapeDtypeStruct((NC, N_ACC, EMB), jnp.float32),
        scratch_types=[
            pltpu.VMEM((2, CH), jnp.int32),
            pltpu.VMEM((2, CH), jnp.int32),
            pltpu.VMEM((2, CH), jnp.int32),
            pltpu.VMEM((CH,), jnp.float32),
            pltpu.VMEM((CH,), jnp.float32),
            pltpu.VMEM((CH,), jnp.float32),
            pltpu.VMEM((CH,), jnp.int32),
            pltpu.VMEM((CH,), jnp.int32),
            pltpu.VMEM((CH,), jnp.int32),
            pltpu.VMEM((3 * CH, EMB), jnp.float32),
            pltpu.SemaphoreType.DMA,
            pltpu.SemaphoreType.DMA,
            pltpu.SemaphoreType.DMA,
            pltpu.VMEM_SHARED((N_ACC, EMB), jnp.float32),
        ],
    )(hyperneigh, meta, w, emb)


def _ln(x, g, b):
    m = jnp.mean(x, axis=-1, keepdims=True)
    v = jnp.mean((x - m) ** 2, axis=-1, keepdims=True)
    return (x - m) * lax.rsqrt(v + 1e-5) * g + b


BLK = 1000


def _tc_body(p0, p1, W1, b1, Wv, bv, Wo, bo, l1g, l1b, Wf1, bf1, Wf2, bf2,
             l2g, l2b, fW, fb, out_ref, acc_ref):
    b = pl.program_id(0)
    agg = p0[0] + p1[0]
    o = jnp.dot(agg, W1[...], preferred_element_type=jnp.float32) + b1[...]
    ct = (((1,), (1,)), ((), ()))
    a = lax.dot_general(o, Wv[...], ct, preferred_element_type=jnp.float32) + bv[...]
    a = lax.dot_general(a, Wo[...], ct, preferred_element_type=jnp.float32) + bo[...]
    y = _ln(o + a, l1g[...], l1b[...])
    f = lax.dot_general(y, Wf1[...], ct, preferred_element_type=jnp.float32) + bf1[...]
    f = jnp.maximum(f, 0.0)
    f = lax.dot_general(f, Wf2[...], ct, preferred_element_type=jnp.float32) + bf2[...]
    z = _ln(y + f, l2g[...], l2b[...])
    colsum = jnp.sum(z, axis=0, keepdims=True)
    coef = jnp.where(b < 4, 1.0 / 4000.0,
                     jnp.where(b < 7, -1.0 / 3000.0, 1.0 / 2000.0))

    @pl.when(b == 0)
    def _():
        acc_ref[...] = jnp.zeros_like(acc_ref)

    acc_ref[...] += coef.astype(jnp.float32) * colsum

    @pl.when(b == (N_USED // BLK) - 1)
    def _():
        h = acc_ref[...]
        r = lax.dot_general(h, fW[...], ct, preferred_element_type=jnp.float32) + fb[...]
        out_ref[...] = jnp.maximum(r, 0.0)[None]


def _tc_dense(partials, W1, b1, Wv, bv, Wo, bo, l1g, l1b, Wf1, bf1, Wf2, bf2,
              l2g, l2b, fW, fb):
    n_blk = N_USED // BLK
    full = lambda arr: pl.BlockSpec(arr.shape, lambda b: (0,) * arr.ndim)
    w_specs = [full(a) for a in (W1, b1, Wv, bv, Wo, bo, l1g, l1b,
                                 Wf1, bf1, Wf2, bf2, l2g, l2b, fW, fb)]
    return pl.pallas_call(
        _tc_body,
        grid=(n_blk,),
        in_specs=[
            pl.BlockSpec((1, BLK, EMB), lambda b: (0, b, 0)),
            pl.BlockSpec((1, BLK, EMB), lambda b: (1, b, 0)),
            *w_specs,
        ],
        out_specs=pl.BlockSpec((1, 1, HID), lambda b: (0, 0, 0)),
        out_shape=jax.ShapeDtypeStruct((1, 1, HID), jnp.float32),
        scratch_shapes=[pltpu.VMEM((1, HID), jnp.float32)],
    )(partials, partials, W1, b1, Wv, bv, Wo, bo, l1g, l1b,
      Wf1, bf1, Wf2, bf2, l2g, l2b, fW, fb)


def _edge_layout(src, dst, w):
    meta = jnp.stack([src.reshape(TOTCH, CH), dst.reshape(TOTCH, CH)],
                     axis=1)
    return meta, w.reshape(TOTCH, CH)


def kernel(hyperneigh, edge_index, edge_weight, acc_len, rej_len, friend_len,
           emb, W1, b1, Wq, bq, Wk, bk, Wv, bv, Wo, bo, ln1_g, ln1_b,
           Wf1, bf1, Wf2, bf2, ln2_g, ln2_b, fc1_W, fc1_b):
    src = edge_index[0]
    dst = edge_index[1]
    pad = E_PAD - src.shape[0]
    src = jnp.concatenate([src, jnp.zeros((pad,), src.dtype)])
    dst = jnp.minimum(dst, N_USED)
    dst = jnp.concatenate([dst, jnp.zeros((pad,), dst.dtype)])
    w = jnp.concatenate([edge_weight, jnp.zeros((pad,), edge_weight.dtype)])
    meta, wm = _edge_layout(src, dst, w)

    partials = _sc_aggregate(hyperneigh, meta, wm, emb)

    r2 = lambda v: v.reshape(1, -1)
    return _tc_dense(partials, W1, r2(b1), Wv, r2(bv), Wo, r2(bo),
                     r2(ln1_g), r2(ln1_b), Wf1, r2(bf1), Wf2, r2(bf2),
                     r2(ln2_g), r2(ln2_b), fc1_W, r2(fc1_b))
